# Initial kernel scaffold; baseline (speedup 1.0000x reference)
import jax
import jax.numpy as jnp
from jax.experimental import pallas as pl


def kernel(word_ids, edge_src, edge_dst, tffrac, sent_raw, embed, W_proj, TF_embed, W_edge, Wk1, Wq1, al1, ar1, w11, b11, w12, b12, g1, be1, Wk2, Wq2, al2, ar2, w21, b21, w22, b22, g2, be2, wh_w, wh_b):
    raise NotImplementedError("write your pallas kernel here")


# speedup vs baseline: 13.2373x; 13.2373x over previous
import functools
import jax
import jax.numpy as jnp
from jax import lax
from jax.experimental import pallas as pl
from jax.experimental.pallas import tpu as pltpu
from jax.experimental.pallas import tpu_sc as plsc

NW, NS, E = 50000, 10000, 320000
D, H, DH, FFN, VOC = 128, 8, 16, 512, 50000
NC, NSUB, NWK = 2, 16, 32
EPW = E // NWK
CE = 2000
NCH = EPW // CE
NS_PAD, NS_SL = 10240, 640
NW_PAD, NW_SL = 50176, 3136
VOC_PAD = 50176
RB = 400
RC = 512
EB = 6400
CW, CW_LAST = 1568, NW - 31 * 1568

_MESH = dict(core_axis_name="c", subcore_axis_name="s",
             num_cores=NC, num_subcores=NSUB)

f32 = jnp.float32
i32 = jnp.int32


def _embed_proj_body(x_ref, wk_ref, wc_ref, o0, o1, o2, o3, ow):
    x = x_ref[...]
    k = jnp.dot(x, wk_ref[...], preferred_element_type=f32)
    for g, o in enumerate((o0, o1, o2, o3)):
        o[...] = k[:, g * 32:(g + 1) * 32]
    ow[...] = jnp.dot(x, wc_ref[...], preferred_element_type=f32)


def _embed_proj(embed, Wk1, Wcat):
    return pl.pallas_call(
        _embed_proj_body,
        grid=(VOC // RB,),
        in_specs=[
            pl.BlockSpec((RB, D), lambda i: (i, 0)),
            pl.BlockSpec((D, D), lambda i: (0, 0)),
            pl.BlockSpec((D, 16), lambda i: (0, 0)),
        ],
        out_specs=[pl.BlockSpec((RB, 32), lambda i: (i, 0))] * 4
        + [pl.BlockSpec((RB, 16), lambda i: (i, 0))],
        out_shape=[jax.ShapeDtypeStruct((VOC, 32), f32)] * 4
        + [jax.ShapeDtypeStruct((VOC, 16), f32)],
    )(embed, Wk1, Wcat)


def _sent_proj_body(x_ref, w_ref, o_ref):
    er = jnp.dot(x_ref[...], w_ref[...], preferred_element_type=f32)
    o_ref[...] = jnp.concatenate([jnp.zeros((RB, 8), f32), er], axis=1)


def _sent_proj(sent_raw, Wpa):
    return pl.pallas_call(
        _sent_proj_body,
        grid=(NS // RB,),
        in_specs=[
            pl.BlockSpec((RB, D), lambda i: (i, 0)),
            pl.BlockSpec((D, 8), lambda i: (0, 0)),
        ],
        out_specs=pl.BlockSpec((RB, 16), lambda i: (i, 0)),
        out_shape=jax.ShapeDtypeStruct((NS, 16), f32),
    )(sent_raw, Wpa)


def _ebias_body(tf_ref, w_ref, o_ref):
    row = tf_ref[...]
    oh = (jnp.broadcast_to(row, (10, EB))
          == lax.broadcasted_iota(i32, (10, EB), 0)).astype(f32)
    o_ref[...] = jnp.dot(w_ref[...], oh, preferred_element_type=f32)


def _ebias(tf2d, tfT):
    return pl.pallas_call(
        _ebias_body,
        grid=(E // EB,),
        in_specs=[
            pl.BlockSpec((1, EB), lambda i: (0, i)),
            pl.BlockSpec((8, 10), lambda i: (0, 0)),
        ],
        out_specs=pl.BlockSpec((8, EB), lambda i: (0, i)),
        out_shape=jax.ShapeDtypeStruct((8, E), f32),
    )(tf2d, tfT)


def _t16_body(x_ref, o_ref):
    o_ref[...] = x_ref[...].T


def _transpose16(x):
    n = x.shape[0]
    return pl.pallas_call(
        _t16_body,
        grid=(n // 128,),
        in_specs=[pl.BlockSpec((128, 16), lambda j: (j, 0))],
        out_specs=pl.BlockSpec((16, 128), lambda j: (0, j)),
        out_shape=jax.ShapeDtypeStruct((16, n), f32),
    )(x)


def _combine_core(np_ref, dp_ref, w1, b1, w2, b2, gg, bb, npass):
    i = pl.program_id(0)
    npb = np_ref[...]
    x = jnp.concatenate([npb[0, p] + npb[1, p] for p in range(npass)], axis=1)
    off = pl.multiple_of(i * RC, 128)
    dp = dp_ref[:, :, pl.ds(off, RC)]
    den = (dp[0] + dp[1]).T
    denb = jnp.broadcast_to(den[:, :, None], (RC, H, DH)).reshape(RC, D)
    a = x / (denb + 1e-9)
    a = jnp.where(a > 0, a, jnp.exp(a) - 1.0)
    hid = jnp.maximum(
        jnp.dot(a, w1[...], preferred_element_type=f32) + b1[...], 0.0)
    h = a + jnp.dot(hid, w2[...], preferred_element_type=f32) + b2[...]
    mu = jnp.mean(h, axis=1, keepdims=True)
    c = h - mu
    var = jnp.mean(c * c, axis=1, keepdims=True)
    return gg[...] * c * lax.rsqrt(var + 1e-6) + bb[...]


def _make_combine_proj_body(npass, nk):
    kw = D // nk

    def body(np_ref, dp_ref, w1, b1, w2, b2, gg, bb, p_ref, *outs):
        st = _combine_core(np_ref, dp_ref, w1, b1, w2, b2, gg, bb, npass)
        y = jnp.dot(st, p_ref[...], preferred_element_type=f32)
        for t in range(nk):
            outs[t][...] = y[:, t * kw:(t + 1) * kw]
        outs[nk][...] = y[:, D:D + 16]

    return body


def _make_combine_head_body(npass):
    def body(np_ref, dp_ref, w1, b1, w2, b2, gg, bb, p_ref, o_ref):
        st = _combine_core(np_ref, dp_ref, w1, b1, w2, b2, gg, bb, npass)
        o_ref[...] = jnp.dot(st, p_ref[...], preferred_element_type=f32)

    return body


def _combine(numpart, denpart, w1, b1, w2, b2, gg, bb, P, nk):
    _, npass, npad, kw = numpart.shape
    n_out = npad
    wspecs = [
        pl.BlockSpec((2, npass, RC, kw), lambda i: (0, 0, i, 0)),
        pl.BlockSpec((2, 8, npad), lambda i: (0, 0, 0)),
        pl.BlockSpec((D, FFN), lambda i: (0, 0)),
        pl.BlockSpec((1, FFN), lambda i: (0, 0)),
        pl.BlockSpec((FFN, D), lambda i: (0, 0)),
        pl.BlockSpec((1, D), lambda i: (0, 0)),
        pl.BlockSpec((1, D), lambda i: (0, 0)),
        pl.BlockSpec((1, D), lambda i: (0, 0)),
        pl.BlockSpec((D, P.shape[1]), lambda i: (0, 0)),
    ]
    if nk == 0:
        out_specs = pl.BlockSpec((RC, 1), lambda i: (i, 0))
        out_shape = jax.ShapeDtypeStruct((n_out, 1), f32)
        body = _make_combine_head_body(npass)
    else:
        kwn = D // nk
        out_specs = ([pl.BlockSpec((RC, kwn), lambda i: (i, 0))] * nk
                     + [pl.BlockSpec((RC, 16), lambda i: (i, 0))])
        out_shape = ([jax.ShapeDtypeStruct((n_out, kwn), f32)] * nk
                     + [jax.ShapeDtypeStruct((n_out, 16), f32)])
        body = _make_combine_proj_body(npass, nk)
    return pl.pallas_call(
        body, grid=(n_out // RC,), in_specs=wspecs,
        out_specs=out_specs, out_shape=out_shape,
    )(numpart, denpart, w1, b1, w2, b2, gg, bb, P)


@functools.lru_cache(maxsize=None)
def _get_sc_word_gather():
    return functools.partial(
        pl.kernel,
        out_type=tuple([jax.ShapeDtypeStruct((NW, 32), f32)] * 4
                       + [jax.ShapeDtypeStruct((16 * NW_PAD,), f32)]),
        mesh=plsc.VectorSubcoreMesh(**_MESH),
        compiler_params=pltpu.CompilerParams(use_tc_tiling_on_sc=False),
        scratch_types=[
            pltpu.VMEM((CW,), i32),
            pltpu.VMEM((CW,), i32),
            pltpu.VMEM((CW, 32), f32),
            pltpu.VMEM((CW,), f32),
        ],
    )(_sc_word_gather_body)


def _sc_word_gather_body(ids_h, kt0, kt1, kt2, kt3, wt_flat,
                         o0, o1, o2, o3, ow, idxv, idxb, kbuf, hbuf):
    wid = lax.axis_index("c") * NSUB + lax.axis_index("s")
    base = wid * CW
    kts = (kt0, kt1, kt2, kt3)
    outs = (o0, o1, o2, o3)

    def run(n):
        pltpu.sync_copy(ids_h.at[pl.ds(base, n)], idxv.at[pl.ds(0, n)])
        for g in range(4):
            pltpu.sync_copy(kts[g].at[idxv], kbuf)
            pltpu.sync_copy(kbuf.at[pl.ds(0, n)], outs[g].at[pl.ds(base, n)])
        for h in range(16):
            def ib(i, _):
                sl = pl.ds(i * 16, 16)
                idxb[sl] = idxv[sl] + h * VOC_PAD
                return 0
            lax.fori_loop(0, CW // 16, ib, 0)
            pltpu.sync_copy(wt_flat.at[idxb], hbuf)
            pltpu.sync_copy(hbuf.at[pl.ds(0, n)],
                            ow.at[pl.ds(h * NW_PAD + base, n)])

    @pl.when(wid < NWK - 1)
    def _():
        run(CW)

    @pl.when(wid == NWK - 1)
    def _():
        for t in range(CW_LAST // 16, CW // 16):
            idxv[pl.ds(t * 16, 16)] = jnp.zeros((16,), i32)
        run(CW_LAST)


@functools.lru_cache(maxsize=None)
def _make_pass_b(nsrc, ndst, npad, sl, hpp, npass):
    kw = hpp * DH

    scratch = [
        pltpu.VMEM((CE,), i32),
        pltpu.VMEM((CE,), i32),
        pltpu.VMEM((CE,), i32),
        pltpu.VMEM((CE,), f32),
        pltpu.VMEM((CE,), f32),
        pltpu.VMEM((CE,), f32),
        pltpu.VMEM((CE,), f32),
        pltpu.VMEM((CE,), f32),
        pltpu.VMEM((CE, kw), f32),
        pltpu.VMEM_SHARED((hpp * nsrc,), f32),
        pltpu.VMEM_SHARED((hpp * ndst,), f32),
        pltpu.VMEM_SHARED((npad, kw), f32),
        pltpu.VMEM_SHARED((npad,), f32),
        pltpu.VMEM_SHARED((npad if hpp == 2 else 16,), f32),
    ]

    def body(elerT_src, elerT_dst, ebt, *rest):
        kts = rest[:npass]
        (src_h, dst_h, z2d, z1d, np_out, dp_out,
         srcv, dstv, idxb, elv, erv, ebv, exv0, exv1, krows,
         elsp, ersp, nsp, d0sp, d1sp) = rest[npass:]
        cid = lax.axis_index("c")
        sid = lax.axis_index("s")
        wid = cid * NSUB + sid
        r0 = sid * sl

        exvs = (exv0, exv1)
        for p in range(npass):
            @pl.when(sid == 0)
            def _():
                pltpu.sync_copy(
                    elerT_src.at[pl.ds(p * hpp * nsrc, hpp * nsrc)], elsp)
                pltpu.sync_copy(
                    elerT_dst.at[pl.ds(8 * ndst + p * hpp * ndst,
                                       hpp * ndst)], ersp)
            pltpu.sync_copy(z2d.at[pl.ds(0, sl), pl.ds(0, kw)],
                            nsp.at[pl.ds(r0, sl)])
            pltpu.sync_copy(z1d.at[pl.ds(0, sl)], d0sp.at[pl.ds(r0, sl)])
            if hpp == 2:
                pltpu.sync_copy(z1d.at[pl.ds(0, sl)], d1sp.at[pl.ds(r0, sl)])
            plsc.subcore_barrier()

            def chunk(ch, _):
                b = wid * EPW + ch * CE
                pltpu.sync_copy(src_h.at[pl.ds(b, CE)], srcv)
                pltpu.sync_copy(dst_h.at[pl.ds(b, CE)], dstv)
                pltpu.sync_copy(kts[p].at[srcv], krows)
                for t in range(hpp):
                    h = p * hpp + t

                    def ib1(i, _):
                        sl16 = pl.ds(i * 16, 16)
                        idxb[sl16] = srcv[sl16] + t * nsrc
                        return 0

                    lax.fori_loop(0, CE // 16, ib1, 0)
                    pltpu.sync_copy(elsp.at[idxb], elv)

                    def ib2(i, _):
                        sl16 = pl.ds(i * 16, 16)
                        idxb[sl16] = dstv[sl16] + t * ndst
                        return 0

                    lax.fori_loop(0, CE // 16, ib2, 0)
                    pltpu.sync_copy(ersp.at[idxb], erv)
                    pltpu.sync_copy(ebt.at[h, pl.ds(b, CE)], ebv)
                    exv = exvs[t]

                    def sb(i, _):
                        sl16 = pl.ds(i * 16, 16)
                        v = elv[sl16] + erv[sl16] + ebv[sl16]
                        v = jnp.maximum(v, 0.2 * v)
                        exv[sl16] = jnp.exp(v)
                        return 0

                    lax.fori_loop(0, CE // 16, sb, 0)

                def mb(i, _):
                    a0 = exv0[pl.ds(i * 16, 16)]
                    if hpp == 2:
                        a1 = exv1[pl.ds(i * 16, 16)]
                    for j in range(16):
                        e = i * 16 + j
                        krows[e, pl.ds(0, 16)] = (
                            krows[e, pl.ds(0, 16)] * a0[j])
                        if hpp == 2:
                            krows[e, pl.ds(16, 16)] = (
                                krows[e, pl.ds(16, 16)] * a1[j])
                    return 0

                lax.fori_loop(0, CE // 16, mb, 0)
                pltpu.sync_copy(krows, nsp.at[dstv], add=True)
                pltpu.sync_copy(exv0, d0sp.at[dstv], add=True)
                if hpp == 2:
                    pltpu.sync_copy(exv1, d1sp.at[dstv], add=True)
                return 0

            lax.fori_loop(0, NCH, chunk, 0)
            plsc.subcore_barrier()
            pltpu.sync_copy(nsp.at[pl.ds(r0, sl)],
                            np_out.at[cid, p, pl.ds(r0, sl)])
            pltpu.sync_copy(d0sp.at[pl.ds(r0, sl)],
                            dp_out.at[cid, p * hpp, pl.ds(r0, sl)])
            if hpp == 2:
                pltpu.sync_copy(d1sp.at[pl.ds(r0, sl)],
                                dp_out.at[cid, p * hpp + 1, pl.ds(r0, sl)])
            plsc.subcore_barrier()

    return functools.partial(
        pl.kernel,
        out_type=(jax.ShapeDtypeStruct((2, npass, npad, kw), f32),
                  jax.ShapeDtypeStruct((2, 8, npad), f32)),
        mesh=plsc.VectorSubcoreMesh(**_MESH),
        compiler_params=pltpu.CompilerParams(use_tc_tiling_on_sc=False),
        scratch_types=scratch,
    )(body)


def _fold(W, a):
    return jnp.sum(W.reshape(D, H, DH) * a[None], axis=-1)


@jax.jit
def kernel(word_ids, edge_src, edge_dst, tffrac, sent_raw, embed, W_proj,
           TF_embed, W_edge, Wk1, Wq1, al1, ar1, w11, b11, w12, b12, g1, be1,
           Wk2, Wq2, al2, ar2, w21, b21, w22, b22, g2, be2, wh_w, wh_b):
    word_ids = word_ids.astype(i32)
    edge_src = edge_src.astype(i32)
    edge_dst = edge_dst.astype(i32)
    tffrac = tffrac.astype(i32)

    AL1, AR1 = _fold(Wk1, al1), _fold(Wq1, ar1)
    AL2, AR2 = _fold(Wk2, al2), _fold(Wq2, ar2)
    tfT = (TF_embed @ W_edge).T
    Wcat0 = jnp.concatenate([AL1, AR2], axis=1)
    Wpa = W_proj @ AR1
    P1 = jnp.concatenate([Wk2, AL2, AR1], axis=1)
    P2 = jnp.concatenate([Wk1, AL1, jnp.zeros((D, 8), f32)], axis=1)
    b11r, b12r = b11.reshape(1, FFN), b12.reshape(1, D)
    b21r, b22r = b21.reshape(1, FFN), b22.reshape(1, D)
    g1r, be1r = g1.reshape(1, D), be1.reshape(1, D)
    g2r, be2r = g2.reshape(1, D), be2.reshape(1, D)
    z2d = jnp.zeros((NW_SL, 32), f32)
    z1d = jnp.zeros((NW_SL,), f32)

    kt0, kt1, kt2, kt3, wt_voc = _embed_proj(embed, Wk1, Wcat0)
    wt_vocT = _transpose16(
        jnp.pad(wt_voc, ((0, VOC_PAD - VOC), (0, 0)))).reshape(-1)
    ser0 = _sent_proj(sent_raw, Wpa)
    selerT0 = _transpose16(
        jnp.pad(ser0, ((0, NS_PAD - NS), (0, 0)))).reshape(-1)
    ebt = _ebias(tffrac.reshape(1, E), tfT)
    k1t0, k1t1, k1t2, k1t3, welerT = _get_sc_word_gather()(
        word_ids, kt0, kt1, kt2, kt3, wt_vocT)

    pb_sent = _make_pass_b(NW_PAD, NS_PAD, NS_PAD, NS_SL, 2, 4)
    pb_word = _make_pass_b(NS_PAD, NW_PAD, NW_PAD, NW_SL, 1, 8)

    np1, dp1 = pb_sent(welerT, selerT0, ebt, k1t0, k1t1, k1t2, k1t3,
                       edge_src, edge_dst, z2d, z1d)
    c1 = _combine(np1, dp1, w11, b11r, w12, b12r, g1r, be1r, P1, nk=8)
    k2t = c1[:8]
    selerT = _transpose16(c1[8]).reshape(-1)

    np2, dp2 = pb_word(selerT, welerT, ebt, *k2t,
                       edge_dst, edge_src, z2d, z1d)
    c2 = _combine(np2, dp2, w21, b21r, w22, b22r, g2r, be2r, P2, nk=4)
    k3t = c2[:4]
    welerT2 = _transpose16(c2[4]).reshape(-1)

    np3, dp3 = pb_sent(welerT2, selerT, ebt, *k3t,
                       edge_src, edge_dst, z2d, z1d)
    result = _combine(np3, dp3, w11, b11r, w12, b12r, g1r, be1r,
                      wh_w, nk=0)
    return result[:NS] + wh_b


# speedup vs baseline: 15.1966x; 1.1480x over previous
import functools
import jax
import jax.numpy as jnp
from jax import lax
from jax.experimental import pallas as pl
from jax.experimental.pallas import tpu as pltpu
from jax.experimental.pallas import tpu_sc as plsc

NW, NS, E = 50000, 10000, 320000
D, H, DH, FFN, VOC = 128, 8, 16, 512, 50000
NC, NSUB, NWK = 2, 16, 32
EPW = E // NWK
CE = 2000
NCH = EPW // CE
NS_PAD, NS_SL = 10240, 640
NW_PAD, NW_SL = 50176, 3136
VOC_PAD = 50176
RB = 400
RC = 512
EB = 6400
CW, CW_LAST = 1568, NW - 31 * 1568

_MESH = dict(core_axis_name="c", subcore_axis_name="s",
             num_cores=NC, num_subcores=NSUB)

f32 = jnp.float32
i32 = jnp.int32


def _embed_proj_body(x_ref, wk_ref, wc_ref, o0, o1, o2, o3, ow):
    x = x_ref[...]
    k = jnp.dot(x, wk_ref[...], preferred_element_type=f32)
    for g, o in enumerate((o0, o1, o2, o3)):
        o[...] = k[:, g * 32:(g + 1) * 32]
    ow[...] = jnp.dot(x, wc_ref[...], preferred_element_type=f32)


def _embed_proj(embed, Wk1, Wcat):
    return pl.pallas_call(
        _embed_proj_body,
        grid=(VOC // RB,),
        in_specs=[
            pl.BlockSpec((RB, D), lambda i: (i, 0)),
            pl.BlockSpec((D, D), lambda i: (0, 0)),
            pl.BlockSpec((D, 16), lambda i: (0, 0)),
        ],
        out_specs=[pl.BlockSpec((RB, 32), lambda i: (i, 0))] * 4
        + [pl.BlockSpec((RB, 16), lambda i: (i, 0))],
        out_shape=[jax.ShapeDtypeStruct((VOC, 32), f32)] * 4
        + [jax.ShapeDtypeStruct((VOC, 16), f32)],
    )(embed, Wk1, Wcat)


def _sent_proj_body(x_ref, w_ref, o_ref):
    er = jnp.dot(x_ref[...], w_ref[...], preferred_element_type=f32)
    o_ref[...] = jnp.concatenate([jnp.zeros((RB, 8), f32), er], axis=1)


def _sent_proj(sent_raw, Wpa):
    return pl.pallas_call(
        _sent_proj_body,
        grid=(NS // RB,),
        in_specs=[
            pl.BlockSpec((RB, D), lambda i: (i, 0)),
            pl.BlockSpec((D, 8), lambda i: (0, 0)),
        ],
        out_specs=pl.BlockSpec((RB, 16), lambda i: (i, 0)),
        out_shape=jax.ShapeDtypeStruct((NS, 16), f32),
    )(sent_raw, Wpa)


def _ebias_body(tf_ref, w_ref, o_ref):
    row = tf_ref[...]
    oh = (jnp.broadcast_to(row, (10, EB))
          == lax.broadcasted_iota(i32, (10, EB), 0)).astype(f32)
    o_ref[...] = jnp.dot(w_ref[...], oh, preferred_element_type=f32)


def _ebias(tf2d, tfT):
    return pl.pallas_call(
        _ebias_body,
        grid=(E // EB,),
        in_specs=[
            pl.BlockSpec((1, EB), lambda i: (0, i)),
            pl.BlockSpec((8, 10), lambda i: (0, 0)),
        ],
        out_specs=pl.BlockSpec((8, EB), lambda i: (0, i)),
        out_shape=jax.ShapeDtypeStruct((8, E), f32),
    )(tf2d, tfT)


def _t16_body(x_ref, o_ref):
    o_ref[...] = x_ref[...].T


def _transpose16(x):
    n = x.shape[0]
    tb = 2048 if n % 2048 == 0 else 1024
    return pl.pallas_call(
        _t16_body,
        grid=(n // tb,),
        in_specs=[pl.BlockSpec((tb, 16), lambda j: (j, 0))],
        out_specs=pl.BlockSpec((16, tb), lambda j: (0, j)),
        out_shape=jax.ShapeDtypeStruct((16, n), f32),
    )(x)


def _combine_core(np_ref, dp_ref, w1, b1, w2, b2, gg, bb, npass):
    i = pl.program_id(0)
    npb = np_ref[...]
    x = jnp.concatenate([npb[0, p] + npb[1, p] for p in range(npass)], axis=1)
    off = pl.multiple_of(i * RC, 128)
    dp = dp_ref[:, :, pl.ds(off, RC)]
    den = (dp[0] + dp[1]).T
    denb = jnp.broadcast_to(den[:, :, None], (RC, H, DH)).reshape(RC, D)
    a = x / (denb + 1e-9)
    a = jnp.where(a > 0, a, jnp.exp(a) - 1.0)
    hid = jnp.maximum(
        jnp.dot(a, w1[...], preferred_element_type=f32) + b1[...], 0.0)
    h = a + jnp.dot(hid, w2[...], preferred_element_type=f32) + b2[...]
    mu = jnp.mean(h, axis=1, keepdims=True)
    c = h - mu
    var = jnp.mean(c * c, axis=1, keepdims=True)
    return gg[...] * c * lax.rsqrt(var + 1e-6) + bb[...]


def _make_combine_proj_body(npass, nk):
    kw = D // nk

    def body(np_ref, dp_ref, w1, b1, w2, b2, gg, bb, p_ref, *outs):
        st = _combine_core(np_ref, dp_ref, w1, b1, w2, b2, gg, bb, npass)
        y = jnp.dot(st, p_ref[...], preferred_element_type=f32)
        for t in range(nk):
            outs[t][...] = y[:, t * kw:(t + 1) * kw]
        outs[nk][...] = y[:, D:D + 16]

    return body


def _make_combine_head_body(npass):
    def body(np_ref, dp_ref, w1, b1, w2, b2, gg, bb, p_ref, o_ref):
        st = _combine_core(np_ref, dp_ref, w1, b1, w2, b2, gg, bb, npass)
        o_ref[...] = jnp.dot(st, p_ref[...], preferred_element_type=f32)

    return body


def _combine(numpart, denpart, w1, b1, w2, b2, gg, bb, P, nk):
    _, npass, npad, kw = numpart.shape
    n_out = npad
    wspecs = [
        pl.BlockSpec((2, npass, RC, kw), lambda i: (0, 0, i, 0)),
        pl.BlockSpec((2, 8, npad), lambda i: (0, 0, 0)),
        pl.BlockSpec((D, FFN), lambda i: (0, 0)),
        pl.BlockSpec((1, FFN), lambda i: (0, 0)),
        pl.BlockSpec((FFN, D), lambda i: (0, 0)),
        pl.BlockSpec((1, D), lambda i: (0, 0)),
        pl.BlockSpec((1, D), lambda i: (0, 0)),
        pl.BlockSpec((1, D), lambda i: (0, 0)),
        pl.BlockSpec((D, P.shape[1]), lambda i: (0, 0)),
    ]
    if nk == 0:
        out_specs = pl.BlockSpec((RC, 1), lambda i: (i, 0))
        out_shape = jax.ShapeDtypeStruct((n_out, 1), f32)
        body = _make_combine_head_body(npass)
    else:
        kwn = D // nk
        out_specs = ([pl.BlockSpec((RC, kwn), lambda i: (i, 0))] * nk
                     + [pl.BlockSpec((RC, 16), lambda i: (i, 0))])
        out_shape = ([jax.ShapeDtypeStruct((n_out, kwn), f32)] * nk
                     + [jax.ShapeDtypeStruct((n_out, 16), f32)])
        body = _make_combine_proj_body(npass, nk)
    return pl.pallas_call(
        body, grid=(n_out // RC,), in_specs=wspecs,
        out_specs=out_specs, out_shape=out_shape,
    )(numpart, denpart, w1, b1, w2, b2, gg, bb, P)


@functools.lru_cache(maxsize=None)
def _get_sc_word_gather():
    return functools.partial(
        pl.kernel,
        out_type=tuple([jax.ShapeDtypeStruct((NW, 32), f32)] * 4
                       + [jax.ShapeDtypeStruct((16 * NW_PAD,), f32)]),
        mesh=plsc.VectorSubcoreMesh(**_MESH),
        compiler_params=pltpu.CompilerParams(use_tc_tiling_on_sc=False),
        scratch_types=[
            pltpu.VMEM((CW,), i32),
            pltpu.VMEM((CW,), i32),
            pltpu.VMEM((CW, 32), f32),
            pltpu.VMEM((CW,), f32),
        ],
    )(_sc_word_gather_body)


def _sc_word_gather_body(ids_h, kt0, kt1, kt2, kt3, wt_flat,
                         o0, o1, o2, o3, ow, idxv, idxb, kbuf, hbuf):
    wid = lax.axis_index("c") * NSUB + lax.axis_index("s")
    base = wid * CW
    kts = (kt0, kt1, kt2, kt3)
    outs = (o0, o1, o2, o3)

    def run(n):
        pltpu.sync_copy(ids_h.at[pl.ds(base, n)], idxv.at[pl.ds(0, n)])
        for g in range(4):
            pltpu.sync_copy(kts[g].at[idxv], kbuf)
            pltpu.sync_copy(kbuf.at[pl.ds(0, n)], outs[g].at[pl.ds(base, n)])
        for h in range(16):
            def ib(i, _):
                sl = pl.ds(i * 16, 16)
                idxb[sl] = idxv[sl] + h * VOC_PAD
                return 0
            lax.fori_loop(0, CW // 16, ib, 0)
            pltpu.sync_copy(wt_flat.at[idxb], hbuf)
            pltpu.sync_copy(hbuf.at[pl.ds(0, n)],
                            ow.at[pl.ds(h * NW_PAD + base, n)])

    @pl.when(wid < NWK - 1)
    def _():
        run(CW)

    @pl.when(wid == NWK - 1)
    def _():
        for t in range(CW_LAST // 16, CW // 16):
            idxv[pl.ds(t * 16, 16)] = jnp.zeros((16,), i32)
        run(CW_LAST)


@functools.lru_cache(maxsize=None)
def _make_pass_b(nsrc, ndst, npad, sl, hpp, npass):
    kw = hpp * DH

    scratch = [
        pltpu.VMEM((CE,), i32),
        pltpu.VMEM((CE,), i32),
        pltpu.VMEM((CE,), i32),
        pltpu.VMEM((CE,), f32),
        pltpu.VMEM((CE,), f32),
        pltpu.VMEM((CE,), f32),
        pltpu.VMEM((CE,), f32),
        pltpu.VMEM((CE,), f32),
        pltpu.VMEM((CE, kw), f32),
        pltpu.VMEM_SHARED((hpp * nsrc,), f32),
        pltpu.VMEM_SHARED((hpp * ndst,), f32),
        pltpu.VMEM_SHARED((npad, kw), f32),
        pltpu.VMEM_SHARED((npad,), f32),
        pltpu.VMEM_SHARED((npad if hpp == 2 else 16,), f32),
    ]

    def body(elerT_src, elerT_dst, ebt, *rest):
        kts = rest[:npass]
        (src_h, dst_h, z2d, z1d, np_out, dp_out,
         srcv, dstv, idxb, elv, erv, ebv, exv0, exv1, krows,
         elsp, ersp, nsp, d0sp, d1sp) = rest[npass:]
        cid = lax.axis_index("c")
        sid = lax.axis_index("s")
        wid = cid * NSUB + sid
        r0 = sid * sl

        exvs = (exv0, exv1)
        for p in range(npass):
            @pl.when(sid == 0)
            def _():
                pltpu.sync_copy(
                    elerT_src.at[pl.ds(p * hpp * nsrc, hpp * nsrc)], elsp)
                pltpu.sync_copy(
                    elerT_dst.at[pl.ds(8 * ndst + p * hpp * ndst,
                                       hpp * ndst)], ersp)
            pltpu.sync_copy(z2d.at[pl.ds(0, sl), pl.ds(0, kw)],
                            nsp.at[pl.ds(r0, sl)])
            pltpu.sync_copy(z1d.at[pl.ds(0, sl)], d0sp.at[pl.ds(r0, sl)])
            if hpp == 2:
                pltpu.sync_copy(z1d.at[pl.ds(0, sl)], d1sp.at[pl.ds(r0, sl)])
            plsc.subcore_barrier()

            def chunk(ch, _):
                b = wid * EPW + ch * CE
                pltpu.sync_copy(src_h.at[pl.ds(b, CE)], srcv)
                pltpu.sync_copy(dst_h.at[pl.ds(b, CE)], dstv)
                pltpu.sync_copy(kts[p].at[srcv], krows)
                for t in range(hpp):
                    h = p * hpp + t

                    def ib1(i, _):
                        sl16 = pl.ds(i * 16, 16)
                        idxb[sl16] = srcv[sl16] + t * nsrc
                        return 0

                    lax.fori_loop(0, CE // 16, ib1, 0)
                    pltpu.sync_copy(elsp.at[idxb], elv)

                    def ib2(i, _):
                        sl16 = pl.ds(i * 16, 16)
                        idxb[sl16] = dstv[sl16] + t * ndst
                        return 0

                    lax.fori_loop(0, CE // 16, ib2, 0)
                    pltpu.sync_copy(ersp.at[idxb], erv)
                    pltpu.sync_copy(ebt.at[h, pl.ds(b, CE)], ebv)
                    exv = exvs[t]

                    def sb(i, _):
                        sl16 = pl.ds(i * 16, 16)
                        v = elv[sl16] + erv[sl16] + ebv[sl16]
                        v = jnp.maximum(v, 0.2 * v)
                        exv[sl16] = jnp.exp(v)
                        return 0

                    lax.fori_loop(0, CE // 16, sb, 0)

                def mb(i, _):
                    a0 = exv0[pl.ds(i * 16, 16)]
                    if hpp == 2:
                        a1 = exv1[pl.ds(i * 16, 16)]
                    for j in range(16):
                        e = i * 16 + j
                        krows[e, pl.ds(0, 16)] = (
                            krows[e, pl.ds(0, 16)] * a0[j])
                        if hpp == 2:
                            krows[e, pl.ds(16, 16)] = (
                                krows[e, pl.ds(16, 16)] * a1[j])
                    return 0

                lax.fori_loop(0, CE // 16, mb, 0)
                pltpu.sync_copy(krows, nsp.at[dstv], add=True)
                pltpu.sync_copy(exv0, d0sp.at[dstv], add=True)
                if hpp == 2:
                    pltpu.sync_copy(exv1, d1sp.at[dstv], add=True)
                return 0

            lax.fori_loop(0, NCH, chunk, 0)
            plsc.subcore_barrier()
            pltpu.sync_copy(nsp.at[pl.ds(r0, sl)],
                            np_out.at[cid, p, pl.ds(r0, sl)])
            pltpu.sync_copy(d0sp.at[pl.ds(r0, sl)],
                            dp_out.at[cid, p * hpp, pl.ds(r0, sl)])
            if hpp == 2:
                pltpu.sync_copy(d1sp.at[pl.ds(r0, sl)],
                                dp_out.at[cid, p * hpp + 1, pl.ds(r0, sl)])
            plsc.subcore_barrier()

    return functools.partial(
        pl.kernel,
        out_type=(jax.ShapeDtypeStruct((2, npass, npad, kw), f32),
                  jax.ShapeDtypeStruct((2, 8, npad), f32)),
        mesh=plsc.VectorSubcoreMesh(**_MESH),
        compiler_params=pltpu.CompilerParams(use_tc_tiling_on_sc=False),
        scratch_types=scratch,
    )(body)


def _fold(W, a):
    return jnp.sum(W.reshape(D, H, DH) * a[None], axis=-1)


@jax.jit
def kernel(word_ids, edge_src, edge_dst, tffrac, sent_raw, embed, W_proj,
           TF_embed, W_edge, Wk1, Wq1, al1, ar1, w11, b11, w12, b12, g1, be1,
           Wk2, Wq2, al2, ar2, w21, b21, w22, b22, g2, be2, wh_w, wh_b):
    word_ids = word_ids.astype(i32)
    edge_src = edge_src.astype(i32)
    edge_dst = edge_dst.astype(i32)
    tffrac = tffrac.astype(i32)

    AL1, AR1 = _fold(Wk1, al1), _fold(Wq1, ar1)
    AL2, AR2 = _fold(Wk2, al2), _fold(Wq2, ar2)
    tfT = (TF_embed @ W_edge).T
    Wcat0 = jnp.concatenate([AL1, AR2], axis=1)
    Wpa = W_proj @ AR1
    P1 = jnp.concatenate([Wk2, AL2, AR1], axis=1)
    P2 = jnp.concatenate([Wk1, AL1, jnp.zeros((D, 8), f32)], axis=1)
    b11r, b12r = b11.reshape(1, FFN), b12.reshape(1, D)
    b21r, b22r = b21.reshape(1, FFN), b22.reshape(1, D)
    g1r, be1r = g1.reshape(1, D), be1.reshape(1, D)
    g2r, be2r = g2.reshape(1, D), be2.reshape(1, D)
    z2d = jnp.zeros((NW_SL, 32), f32)
    z1d = jnp.zeros((NW_SL,), f32)

    kt0, kt1, kt2, kt3, wt_voc = _embed_proj(embed, Wk1, Wcat0)
    wt_vocT = _transpose16(
        jnp.pad(wt_voc, ((0, VOC_PAD - VOC), (0, 0)))).reshape(-1)
    ser0 = _sent_proj(sent_raw, Wpa)
    selerT0 = _transpose16(
        jnp.pad(ser0, ((0, NS_PAD - NS), (0, 0)))).reshape(-1)
    ebt = _ebias(tffrac.reshape(1, E), tfT)
    k1t0, k1t1, k1t2, k1t3, welerT = _get_sc_word_gather()(
        word_ids, kt0, kt1, kt2, kt3, wt_vocT)

    pb_sent = _make_pass_b(NW_PAD, NS_PAD, NS_PAD, NS_SL, 2, 4)
    pb_word = _make_pass_b(NS_PAD, NW_PAD, NW_PAD, NW_SL, 1, 8)

    np1, dp1 = pb_sent(welerT, selerT0, ebt, k1t0, k1t1, k1t2, k1t3,
                       edge_src, edge_dst, z2d, z1d)
    c1 = _combine(np1, dp1, w11, b11r, w12, b12r, g1r, be1r, P1, nk=8)
    k2t = c1[:8]
    selerT = _transpose16(c1[8]).reshape(-1)

    np2, dp2 = pb_word(selerT, welerT, ebt, *k2t,
                       edge_dst, edge_src, z2d, z1d)
    c2 = _combine(np2, dp2, w21, b21r, w22, b22r, g2r, be2r, P2, nk=4)
    k3t = c2[:4]
    welerT2 = _transpose16(c2[4]).reshape(-1)

    np3, dp3 = pb_sent(welerT2, selerT, ebt, *k3t,
                       edge_src, edge_dst, z2d, z1d)
    result = _combine(np3, dp3, w11, b11r, w12, b12r, g1r, be1r,
                      wh_w, nk=0)
    return result[:NS] + wh_b


# speedup vs baseline: 16.5347x; 1.0881x over previous
import functools
import jax
import jax.numpy as jnp
from jax import lax
from jax.experimental import pallas as pl
from jax.experimental.pallas import tpu as pltpu
from jax.experimental.pallas import tpu_sc as plsc

NW, NS, E = 50000, 10000, 320000
D, H, DH, FFN, VOC = 128, 8, 16, 512, 50000
NC, NSUB, NWK = 2, 16, 32
EPW = E // NWK
CE = 2000
NCH = EPW // CE
NS_PAD, NS_SL = 10240, 640
NW_PAD, NW_SL = 50176, 3136
VOC_PAD = 50176
RB = 400
RC = 512
EB = 6400
CW, CW_LAST = 1568, NW - 31 * 1568

_MESH = dict(core_axis_name="c", subcore_axis_name="s",
             num_cores=NC, num_subcores=NSUB)

f32 = jnp.float32
i32 = jnp.int32


def _embed_proj_body(x_ref, wk_ref, wc_ref, o0, o1, o2, o3, ow):
    x = x_ref[...]
    k = jnp.dot(x, wk_ref[...], preferred_element_type=f32)
    for g, o in enumerate((o0, o1, o2, o3)):
        o[...] = k[:, g * 32:(g + 1) * 32]
    ow[...] = jnp.dot(x, wc_ref[...], preferred_element_type=f32)


def _embed_proj(embed, Wk1, Wcat):
    return pl.pallas_call(
        _embed_proj_body,
        grid=(VOC // RB,),
        in_specs=[
            pl.BlockSpec((RB, D), lambda i: (i, 0)),
            pl.BlockSpec((D, D), lambda i: (0, 0)),
            pl.BlockSpec((D, 16), lambda i: (0, 0)),
        ],
        out_specs=[pl.BlockSpec((RB, 32), lambda i: (i, 0))] * 4
        + [pl.BlockSpec((RB, 16), lambda i: (i, 0))],
        out_shape=[jax.ShapeDtypeStruct((VOC, 32), f32)] * 4
        + [jax.ShapeDtypeStruct((VOC, 16), f32)],
    )(embed, Wk1, Wcat)


def _sent_proj_body(x_ref, w_ref, o_ref):
    er = jnp.dot(x_ref[...], w_ref[...], preferred_element_type=f32)
    o_ref[...] = jnp.concatenate([jnp.zeros((RB, 8), f32), er], axis=1)


def _sent_proj(sent_raw, Wpa):
    return pl.pallas_call(
        _sent_proj_body,
        grid=(NS // RB,),
        in_specs=[
            pl.BlockSpec((RB, D), lambda i: (i, 0)),
            pl.BlockSpec((D, 8), lambda i: (0, 0)),
        ],
        out_specs=pl.BlockSpec((RB, 16), lambda i: (i, 0)),
        out_shape=jax.ShapeDtypeStruct((NS, 16), f32),
    )(sent_raw, Wpa)


def _ebias_body(tf_ref, w_ref, o_ref):
    row = tf_ref[...]
    oh = (jnp.broadcast_to(row, (10, EB))
          == lax.broadcasted_iota(i32, (10, EB), 0)).astype(f32)
    o_ref[...] = jnp.dot(w_ref[...], oh, preferred_element_type=f32)


def _ebias(tf2d, tfT):
    return pl.pallas_call(
        _ebias_body,
        grid=(E // EB,),
        in_specs=[
            pl.BlockSpec((1, EB), lambda i: (0, i)),
            pl.BlockSpec((8, 10), lambda i: (0, 0)),
        ],
        out_specs=pl.BlockSpec((8, EB), lambda i: (0, i)),
        out_shape=jax.ShapeDtypeStruct((8, E), f32),
    )(tf2d, tfT)


def _t16_body(x_ref, o_ref):
    o_ref[...] = x_ref[...].T


def _transpose16(x):
    n = x.shape[0]
    tb = 2048 if n % 2048 == 0 else 1024
    return pl.pallas_call(
        _t16_body,
        grid=(n // tb,),
        in_specs=[pl.BlockSpec((tb, 16), lambda j: (j, 0))],
        out_specs=pl.BlockSpec((16, tb), lambda j: (0, j)),
        out_shape=jax.ShapeDtypeStruct((16, n), f32),
    )(x)


def _combine_core(np_ref, dp_ref, w1, b1, w2, b2, gg, bb, npass):
    i = pl.program_id(0)
    npb = np_ref[...]
    x = npb[0] + npb[1]
    off = pl.multiple_of(i * RC, 128)
    dp = dp_ref[:, :, pl.ds(off, RC)]
    den = (dp[0] + dp[1]).T
    denb = jnp.broadcast_to(den[:, :, None], (RC, H, DH)).reshape(RC, D)
    a = x / (denb + 1e-9)
    a = jnp.where(a > 0, a, jnp.exp(a) - 1.0)
    hid = jnp.maximum(
        jnp.dot(a, w1[...], preferred_element_type=f32) + b1[...], 0.0)
    h = a + jnp.dot(hid, w2[...], preferred_element_type=f32) + b2[...]
    mu = jnp.mean(h, axis=1, keepdims=True)
    c = h - mu
    var = jnp.mean(c * c, axis=1, keepdims=True)
    return gg[...] * c * lax.rsqrt(var + 1e-6) + bb[...]


def _make_combine_proj_body(npass, nk):
    kw = D // nk

    def body(np_ref, dp_ref, w1, b1, w2, b2, gg, bb, p_ref, *outs):
        st = _combine_core(np_ref, dp_ref, w1, b1, w2, b2, gg, bb, npass)
        y = jnp.dot(st, p_ref[...], preferred_element_type=f32)
        for t in range(nk):
            outs[t][...] = y[:, t * kw:(t + 1) * kw]
        outs[nk][...] = y[:, D:D + 16]

    return body


def _make_combine_head_body(npass):
    def body(np_ref, dp_ref, w1, b1, w2, b2, gg, bb, p_ref, o_ref):
        st = _combine_core(np_ref, dp_ref, w1, b1, w2, b2, gg, bb, npass)
        o_ref[...] = jnp.dot(st, p_ref[...], preferred_element_type=f32)

    return body


def _combine(numpart, denpart, w1, b1, w2, b2, gg, bb, P, nk):
    _, npad, _ = numpart.shape
    npass = 0
    n_out = npad
    wspecs = [
        pl.BlockSpec((2, RC, D), lambda i: (0, i, 0)),
        pl.BlockSpec((2, 8, npad), lambda i: (0, 0, 0)),
        pl.BlockSpec((D, FFN), lambda i: (0, 0)),
        pl.BlockSpec((1, FFN), lambda i: (0, 0)),
        pl.BlockSpec((FFN, D), lambda i: (0, 0)),
        pl.BlockSpec((1, D), lambda i: (0, 0)),
        pl.BlockSpec((1, D), lambda i: (0, 0)),
        pl.BlockSpec((1, D), lambda i: (0, 0)),
        pl.BlockSpec((D, P.shape[1]), lambda i: (0, 0)),
    ]
    if nk == 0:
        out_specs = pl.BlockSpec((RC, 1), lambda i: (i, 0))
        out_shape = jax.ShapeDtypeStruct((n_out, 1), f32)
        body = _make_combine_head_body(npass)
    else:
        kwn = D // nk
        out_specs = ([pl.BlockSpec((RC, kwn), lambda i: (i, 0))] * nk
                     + [pl.BlockSpec((RC, 16), lambda i: (i, 0))])
        out_shape = ([jax.ShapeDtypeStruct((n_out, kwn), f32)] * nk
                     + [jax.ShapeDtypeStruct((n_out, 16), f32)])
        body = _make_combine_proj_body(npass, nk)
    return pl.pallas_call(
        body, grid=(n_out // RC,), in_specs=wspecs,
        out_specs=out_specs, out_shape=out_shape,
    )(numpart, denpart, w1, b1, w2, b2, gg, bb, P)


@functools.lru_cache(maxsize=None)
def _get_sc_word_gather():
    return functools.partial(
        pl.kernel,
        out_type=tuple([jax.ShapeDtypeStruct((NW, 32), f32)] * 4
                       + [jax.ShapeDtypeStruct((16 * NW_PAD,), f32)]),
        mesh=plsc.VectorSubcoreMesh(**_MESH),
        compiler_params=pltpu.CompilerParams(use_tc_tiling_on_sc=False),
        scratch_types=[
            pltpu.VMEM((CW,), i32),
            pltpu.VMEM((CW,), i32),
            pltpu.VMEM((CW, 32), f32),
            pltpu.VMEM((CW,), f32),
        ],
    )(_sc_word_gather_body)


def _sc_word_gather_body(ids_h, kt0, kt1, kt2, kt3, wt_flat,
                         o0, o1, o2, o3, ow, idxv, idxb, kbuf, hbuf):
    wid = lax.axis_index("c") * NSUB + lax.axis_index("s")
    base = wid * CW
    kts = (kt0, kt1, kt2, kt3)
    outs = (o0, o1, o2, o3)

    def run(n):
        pltpu.sync_copy(ids_h.at[pl.ds(base, n)], idxv.at[pl.ds(0, n)])
        for g in range(4):
            pltpu.sync_copy(kts[g].at[idxv], kbuf)
            pltpu.sync_copy(kbuf.at[pl.ds(0, n)], outs[g].at[pl.ds(base, n)])
        for h in range(16):
            def ib(i, _):
                sl = pl.ds(i * 16, 16)
                idxb[sl] = idxv[sl] + h * VOC_PAD
                return 0
            lax.fori_loop(0, CW // 16, ib, 0)
            pltpu.sync_copy(wt_flat.at[idxb], hbuf)
            pltpu.sync_copy(hbuf.at[pl.ds(0, n)],
                            ow.at[pl.ds(h * NW_PAD + base, n)])

    @pl.when(wid < NWK - 1)
    def _():
        run(CW)

    @pl.when(wid == NWK - 1)
    def _():
        for t in range(CW_LAST // 16, CW // 16):
            idxv[pl.ds(t * 16, 16)] = jnp.zeros((16,), i32)
        run(CW_LAST)


@functools.lru_cache(maxsize=None)
def _make_pass_b(nsrc, ndst, npad, sl, hpp, npass):
    kw = hpp * DH

    scratch = [
        pltpu.VMEM((CE,), i32),
        pltpu.VMEM((CE,), i32),
        pltpu.VMEM((CE,), i32),
        pltpu.VMEM((CE,), f32),
        pltpu.VMEM((CE,), f32),
        pltpu.VMEM((CE,), f32),
        pltpu.VMEM((CE,), f32),
        pltpu.VMEM((CE,), f32),
        pltpu.VMEM((CE, kw), f32),
        pltpu.VMEM_SHARED((hpp * nsrc,), f32),
        pltpu.VMEM_SHARED((hpp * ndst,), f32),
        pltpu.VMEM_SHARED((npad, kw), f32),
        pltpu.VMEM_SHARED((npad,), f32),
        pltpu.VMEM_SHARED((npad if hpp == 2 else 16,), f32),
    ]

    def body(elerT_src, elerT_dst, ebt, *rest):
        kts = rest[:npass]
        (src_h, dst_h, z2d, z1d, np_out, dp_out,
         srcv, dstv, idxb, elv, erv, ebv, exv0, exv1, krows,
         elsp, ersp, nsp, d0sp, d1sp) = rest[npass:]
        cid = lax.axis_index("c")
        sid = lax.axis_index("s")
        wid = cid * NSUB + sid
        r0 = sid * sl

        exvs = (exv0, exv1)
        for p in range(npass):
            @pl.when(sid == 0)
            def _():
                pltpu.sync_copy(
                    elerT_src.at[pl.ds(p * hpp * nsrc, hpp * nsrc)], elsp)
                pltpu.sync_copy(
                    elerT_dst.at[pl.ds(8 * ndst + p * hpp * ndst,
                                       hpp * ndst)], ersp)
            pltpu.sync_copy(z2d.at[pl.ds(0, sl), pl.ds(0, kw)],
                            nsp.at[pl.ds(r0, sl)])
            pltpu.sync_copy(z1d.at[pl.ds(0, sl)], d0sp.at[pl.ds(r0, sl)])
            if hpp == 2:
                pltpu.sync_copy(z1d.at[pl.ds(0, sl)], d1sp.at[pl.ds(r0, sl)])
            plsc.subcore_barrier()

            def chunk(ch, _):
                b = wid * EPW + ch * CE
                pltpu.sync_copy(src_h.at[pl.ds(b, CE)], srcv)
                pltpu.sync_copy(dst_h.at[pl.ds(b, CE)], dstv)
                pltpu.sync_copy(kts[p].at[srcv], krows)
                for t in range(hpp):
                    h = p * hpp + t

                    def ib1(i, _):
                        sl16 = pl.ds(i * 16, 16)
                        idxb[sl16] = srcv[sl16] + t * nsrc
                        return 0

                    lax.fori_loop(0, CE // 16, ib1, 0)
                    pltpu.sync_copy(elsp.at[idxb], elv)

                    def ib2(i, _):
                        sl16 = pl.ds(i * 16, 16)
                        idxb[sl16] = dstv[sl16] + t * ndst
                        return 0

                    lax.fori_loop(0, CE // 16, ib2, 0)
                    pltpu.sync_copy(ersp.at[idxb], erv)
                    pltpu.sync_copy(ebt.at[h, pl.ds(b, CE)], ebv)
                    exv = exvs[t]

                    def sb(i, _):
                        sl16 = pl.ds(i * 16, 16)
                        v = elv[sl16] + erv[sl16] + ebv[sl16]
                        v = jnp.maximum(v, 0.2 * v)
                        exv[sl16] = jnp.exp(v)
                        return 0

                    lax.fori_loop(0, CE // 16, sb, 0)

                def mb(i, _):
                    a0 = exv0[pl.ds(i * 16, 16)]
                    if hpp == 2:
                        a1 = exv1[pl.ds(i * 16, 16)]
                    for j in range(16):
                        e = i * 16 + j
                        krows[e, pl.ds(0, 16)] = (
                            krows[e, pl.ds(0, 16)] * a0[j])
                        if hpp == 2:
                            krows[e, pl.ds(16, 16)] = (
                                krows[e, pl.ds(16, 16)] * a1[j])
                    return 0

                lax.fori_loop(0, CE // 16, mb, 0)
                pltpu.sync_copy(krows, nsp.at[dstv], add=True)
                pltpu.sync_copy(exv0, d0sp.at[dstv], add=True)
                if hpp == 2:
                    pltpu.sync_copy(exv1, d1sp.at[dstv], add=True)
                return 0

            lax.fori_loop(0, NCH, chunk, 0)
            plsc.subcore_barrier()
            pltpu.sync_copy(nsp.at[pl.ds(r0, sl)],
                            np_out.at[cid, pl.ds(r0, sl),
                                      pl.ds(p * kw, kw)])
            pltpu.sync_copy(d0sp.at[pl.ds(r0, sl)],
                            dp_out.at[cid, p * hpp, pl.ds(r0, sl)])
            if hpp == 2:
                pltpu.sync_copy(d1sp.at[pl.ds(r0, sl)],
                                dp_out.at[cid, p * hpp + 1, pl.ds(r0, sl)])
            plsc.subcore_barrier()

    return functools.partial(
        pl.kernel,
        out_type=(jax.ShapeDtypeStruct((2, npad, D), f32),
                  jax.ShapeDtypeStruct((2, 8, npad), f32)),
        mesh=plsc.VectorSubcoreMesh(**_MESH),
        compiler_params=pltpu.CompilerParams(use_tc_tiling_on_sc=False),
        scratch_types=scratch,
    )(body)


def _fold(W, a):
    return jnp.sum(W.reshape(D, H, DH) * a[None], axis=-1)


@jax.jit
def kernel(word_ids, edge_src, edge_dst, tffrac, sent_raw, embed, W_proj,
           TF_embed, W_edge, Wk1, Wq1, al1, ar1, w11, b11, w12, b12, g1, be1,
           Wk2, Wq2, al2, ar2, w21, b21, w22, b22, g2, be2, wh_w, wh_b):
    word_ids = word_ids.astype(i32)
    edge_src = edge_src.astype(i32)
    edge_dst = edge_dst.astype(i32)
    tffrac = tffrac.astype(i32)

    AL1, AR1 = _fold(Wk1, al1), _fold(Wq1, ar1)
    AL2, AR2 = _fold(Wk2, al2), _fold(Wq2, ar2)
    tfT = (TF_embed @ W_edge).T
    Wcat0 = jnp.concatenate([AL1, AR2], axis=1)
    Wpa = W_proj @ AR1
    P1 = jnp.concatenate([Wk2, AL2, AR1], axis=1)
    P2 = jnp.concatenate([Wk1, AL1, jnp.zeros((D, 8), f32)], axis=1)
    b11r, b12r = b11.reshape(1, FFN), b12.reshape(1, D)
    b21r, b22r = b21.reshape(1, FFN), b22.reshape(1, D)
    g1r, be1r = g1.reshape(1, D), be1.reshape(1, D)
    g2r, be2r = g2.reshape(1, D), be2.reshape(1, D)
    z2d = jnp.zeros((NW_SL, 32), f32)
    z1d = jnp.zeros((NW_SL,), f32)

    kt0, kt1, kt2, kt3, wt_voc = _embed_proj(embed, Wk1, Wcat0)
    wt_vocT = _transpose16(
        jnp.pad(wt_voc, ((0, VOC_PAD - VOC), (0, 0)))).reshape(-1)
    ser0 = _sent_proj(sent_raw, Wpa)
    selerT0 = _transpose16(
        jnp.pad(ser0, ((0, NS_PAD - NS), (0, 0)))).reshape(-1)
    ebt = _ebias(tffrac.reshape(1, E), tfT)
    k1t0, k1t1, k1t2, k1t3, welerT = _get_sc_word_gather()(
        word_ids, kt0, kt1, kt2, kt3, wt_vocT)

    pb_sent = _make_pass_b(NW_PAD, NS_PAD, NS_PAD, NS_SL, 2, 4)
    pb_word = _make_pass_b(NS_PAD, NW_PAD, NW_PAD, NW_SL, 1, 8)

    np1, dp1 = pb_sent(welerT, selerT0, ebt, k1t0, k1t1, k1t2, k1t3,
                       edge_src, edge_dst, z2d, z1d)
    c1 = _combine(np1, dp1, w11, b11r, w12, b12r, g1r, be1r, P1, nk=8)
    k2t = c1[:8]
    selerT = _transpose16(c1[8]).reshape(-1)

    np2, dp2 = pb_word(selerT, welerT, ebt, *k2t,
                       edge_dst, edge_src, z2d, z1d)
    c2 = _combine(np2, dp2, w21, b21r, w22, b22r, g2r, be2r, P2, nk=4)
    k3t = c2[:4]
    welerT2 = _transpose16(c2[4]).reshape(-1)

    np3, dp3 = pb_sent(welerT2, selerT, ebt, *k3t,
                       edge_src, edge_dst, z2d, z1d)
    result = _combine(np3, dp3, w11, b11r, w12, b12r, g1r, be1r,
                      wh_w, nk=0)
    return result[:NS] + wh_b


# speedup vs baseline: 16.5622x; 1.0017x over previous
import functools
import jax
import jax.numpy as jnp
from jax import lax
from jax.experimental import pallas as pl
from jax.experimental.pallas import tpu as pltpu
from jax.experimental.pallas import tpu_sc as plsc

NW, NS, E = 50000, 10000, 320000
D, H, DH, FFN, VOC = 128, 8, 16, 512, 50000
NC, NSUB, NWK = 2, 16, 32
EPW = E // NWK
CE = 2000
NCH = EPW // CE
NS_PAD, NS_SL = 10240, 640
NW_PAD, NW_SL = 50176, 3136
VOC_PAD = 50176
RB = 400
RC = 512
EB = 6400
CW, CW_LAST = 1568, NW - 31 * 1568

_MESH = dict(core_axis_name="c", subcore_axis_name="s",
             num_cores=NC, num_subcores=NSUB)

f32 = jnp.float32
i32 = jnp.int32


def _embed_proj_body(x_ref, wk_ref, wc_ref, o0, o1, o2, o3, ow):
    x = x_ref[...]
    k = jnp.dot(x, wk_ref[...], preferred_element_type=f32)
    for g, o in enumerate((o0, o1, o2, o3)):
        o[...] = k[:, g * 32:(g + 1) * 32]
    ow[...] = jnp.dot(x, wc_ref[...], preferred_element_type=f32)


def _embed_proj(embed, Wk1, Wcat):
    return pl.pallas_call(
        _embed_proj_body,
        grid=(VOC // RB,),
        in_specs=[
            pl.BlockSpec((RB, D), lambda i: (i, 0)),
            pl.BlockSpec((D, D), lambda i: (0, 0)),
            pl.BlockSpec((D, 16), lambda i: (0, 0)),
        ],
        out_specs=[pl.BlockSpec((RB, 32), lambda i: (i, 0))] * 4
        + [pl.BlockSpec((RB, 16), lambda i: (i, 0))],
        out_shape=[jax.ShapeDtypeStruct((VOC, 32), f32)] * 4
        + [jax.ShapeDtypeStruct((VOC, 16), f32)],
    )(embed, Wk1, Wcat)


def _sent_proj_body(x_ref, w_ref, o_ref):
    er = jnp.dot(x_ref[...], w_ref[...], preferred_element_type=f32)
    o_ref[...] = jnp.concatenate([jnp.zeros((RB, 8), f32), er], axis=1)


def _sent_proj(sent_raw, Wpa):
    return pl.pallas_call(
        _sent_proj_body,
        grid=(NS // RB,),
        in_specs=[
            pl.BlockSpec((RB, D), lambda i: (i, 0)),
            pl.BlockSpec((D, 8), lambda i: (0, 0)),
        ],
        out_specs=pl.BlockSpec((RB, 16), lambda i: (i, 0)),
        out_shape=jax.ShapeDtypeStruct((NS, 16), f32),
    )(sent_raw, Wpa)


def _ebias_body(tf_ref, w_ref, o_ref):
    row = tf_ref[...]
    oh = (jnp.broadcast_to(row, (10, EB))
          == lax.broadcasted_iota(i32, (10, EB), 0)).astype(f32)
    o_ref[...] = jnp.dot(w_ref[...], oh, preferred_element_type=f32)


def _ebias(tf2d, tfT):
    return pl.pallas_call(
        _ebias_body,
        grid=(E // EB,),
        in_specs=[
            pl.BlockSpec((1, EB), lambda i: (0, i)),
            pl.BlockSpec((8, 10), lambda i: (0, 0)),
        ],
        out_specs=pl.BlockSpec((8, EB), lambda i: (0, i)),
        out_shape=jax.ShapeDtypeStruct((8, E), f32),
    )(tf2d, tfT)


def _t16_body(x_ref, o_ref):
    o_ref[...] = x_ref[...].T


def _transpose16(x):
    n = x.shape[0]
    tb = 2048 if n % 2048 == 0 else 1024
    return pl.pallas_call(
        _t16_body,
        grid=(n // tb,),
        in_specs=[pl.BlockSpec((tb, 16), lambda j: (j, 0))],
        out_specs=pl.BlockSpec((16, tb), lambda j: (0, j)),
        out_shape=jax.ShapeDtypeStruct((16, n), f32),
    )(x)


def _combine_core(np_ref, dp_ref, w1, b1, w2, b2, gg, bb, npass):
    i = pl.program_id(0)
    npb = np_ref[...]
    x = npb[0] + npb[1]
    off = pl.multiple_of(i * RC, 128)
    dp = dp_ref[:, :, pl.ds(off, RC)]
    den = (dp[0] + dp[1]).T
    denb = jnp.broadcast_to(den[:, :, None], (RC, H, DH)).reshape(RC, D)
    a = x / (denb + 1e-9)
    a = jnp.where(a > 0, a, jnp.exp(a) - 1.0)
    bf = jnp.bfloat16
    hid = jnp.maximum(
        jnp.dot(a.astype(bf), w1[...].astype(bf),
                preferred_element_type=f32) + b1[...], 0.0)
    h = a + jnp.dot(hid.astype(bf), w2[...].astype(bf),
                    preferred_element_type=f32) + b2[...]
    mu = jnp.mean(h, axis=1, keepdims=True)
    c = h - mu
    var = jnp.mean(c * c, axis=1, keepdims=True)
    return gg[...] * c * lax.rsqrt(var + 1e-6) + bb[...]


def _make_combine_proj_body(npass, nk):
    kw = D // nk

    def body(np_ref, dp_ref, w1, b1, w2, b2, gg, bb, p_ref, *outs):
        st = _combine_core(np_ref, dp_ref, w1, b1, w2, b2, gg, bb, npass)
        y = jnp.dot(st, p_ref[...], preferred_element_type=f32)
        for t in range(nk):
            outs[t][...] = y[:, t * kw:(t + 1) * kw]
        outs[nk][...] = y[:, D:D + 16]

    return body


def _make_combine_head_body(npass):
    def body(np_ref, dp_ref, w1, b1, w2, b2, gg, bb, p_ref, o_ref):
        st = _combine_core(np_ref, dp_ref, w1, b1, w2, b2, gg, bb, npass)
        o_ref[...] = jnp.dot(st, p_ref[...], preferred_element_type=f32)

    return body


def _combine(numpart, denpart, w1, b1, w2, b2, gg, bb, P, nk):
    _, npad, _ = numpart.shape
    npass = 0
    n_out = npad
    wspecs = [
        pl.BlockSpec((2, RC, D), lambda i: (0, i, 0)),
        pl.BlockSpec((2, 8, npad), lambda i: (0, 0, 0)),
        pl.BlockSpec((D, FFN), lambda i: (0, 0)),
        pl.BlockSpec((1, FFN), lambda i: (0, 0)),
        pl.BlockSpec((FFN, D), lambda i: (0, 0)),
        pl.BlockSpec((1, D), lambda i: (0, 0)),
        pl.BlockSpec((1, D), lambda i: (0, 0)),
        pl.BlockSpec((1, D), lambda i: (0, 0)),
        pl.BlockSpec((D, P.shape[1]), lambda i: (0, 0)),
    ]
    if nk == 0:
        out_specs = pl.BlockSpec((RC, 1), lambda i: (i, 0))
        out_shape = jax.ShapeDtypeStruct((n_out, 1), f32)
        body = _make_combine_head_body(npass)
    else:
        kwn = D // nk
        out_specs = ([pl.BlockSpec((RC, kwn), lambda i: (i, 0))] * nk
                     + [pl.BlockSpec((RC, 16), lambda i: (i, 0))])
        out_shape = ([jax.ShapeDtypeStruct((n_out, kwn), f32)] * nk
                     + [jax.ShapeDtypeStruct((n_out, 16), f32)])
        body = _make_combine_proj_body(npass, nk)
    return pl.pallas_call(
        body, grid=(n_out // RC,), in_specs=wspecs,
        out_specs=out_specs, out_shape=out_shape,
    )(numpart, denpart, w1, b1, w2, b2, gg, bb, P)


@functools.lru_cache(maxsize=None)
def _get_sc_word_gather():
    return functools.partial(
        pl.kernel,
        out_type=tuple([jax.ShapeDtypeStruct((NW, 32), f32)] * 4
                       + [jax.ShapeDtypeStruct((16 * NW_PAD,), f32)]),
        mesh=plsc.VectorSubcoreMesh(**_MESH),
        compiler_params=pltpu.CompilerParams(use_tc_tiling_on_sc=False),
        scratch_types=[
            pltpu.VMEM((CW,), i32),
            pltpu.VMEM((CW,), i32),
            pltpu.VMEM((CW, 32), f32),
            pltpu.VMEM((CW,), f32),
        ],
    )(_sc_word_gather_body)


def _sc_word_gather_body(ids_h, kt0, kt1, kt2, kt3, wt_flat,
                         o0, o1, o2, o3, ow, idxv, idxb, kbuf, hbuf):
    wid = lax.axis_index("c") * NSUB + lax.axis_index("s")
    base = wid * CW
    kts = (kt0, kt1, kt2, kt3)
    outs = (o0, o1, o2, o3)

    def run(n):
        pltpu.sync_copy(ids_h.at[pl.ds(base, n)], idxv.at[pl.ds(0, n)])
        for g in range(4):
            pltpu.sync_copy(kts[g].at[idxv], kbuf)
            pltpu.sync_copy(kbuf.at[pl.ds(0, n)], outs[g].at[pl.ds(base, n)])
        for h in range(16):
            def ib(i, _):
                sl = pl.ds(i * 16, 16)
                idxb[sl] = idxv[sl] + h * VOC_PAD
                return 0
            lax.fori_loop(0, CW // 16, ib, 0)
            pltpu.sync_copy(wt_flat.at[idxb], hbuf)
            pltpu.sync_copy(hbuf.at[pl.ds(0, n)],
                            ow.at[pl.ds(h * NW_PAD + base, n)])

    @pl.when(wid < NWK - 1)
    def _():
        run(CW)

    @pl.when(wid == NWK - 1)
    def _():
        for t in range(CW_LAST // 16, CW // 16):
            idxv[pl.ds(t * 16, 16)] = jnp.zeros((16,), i32)
        run(CW_LAST)


@functools.lru_cache(maxsize=None)
def _make_pass_b(nsrc, ndst, npad, sl, hpp, npass):
    kw = hpp * DH

    scratch = [
        pltpu.VMEM((CE,), i32),
        pltpu.VMEM((CE,), i32),
        pltpu.VMEM((CE,), i32),
        pltpu.VMEM((CE,), f32),
        pltpu.VMEM((CE,), f32),
        pltpu.VMEM((CE,), f32),
        pltpu.VMEM((CE,), f32),
        pltpu.VMEM((CE,), f32),
        pltpu.VMEM((CE, kw), f32),
        pltpu.VMEM_SHARED((hpp * nsrc,), f32),
        pltpu.VMEM_SHARED((hpp * ndst,), f32),
        pltpu.VMEM_SHARED((npad, kw), f32),
        pltpu.VMEM_SHARED((npad,), f32),
        pltpu.VMEM_SHARED((npad if hpp == 2 else 16,), f32),
    ]

    def body(elerT_src, elerT_dst, ebt, *rest):
        kts = rest[:npass]
        (src_h, dst_h, z2d, z1d, np_out, dp_out,
         srcv, dstv, idxb, elv, erv, ebv, exv0, exv1, krows,
         elsp, ersp, nsp, d0sp, d1sp) = rest[npass:]
        cid = lax.axis_index("c")
        sid = lax.axis_index("s")
        wid = cid * NSUB + sid
        r0 = sid * sl

        exvs = (exv0, exv1)
        for p in range(npass):
            @pl.when(sid == 0)
            def _():
                pltpu.sync_copy(
                    elerT_src.at[pl.ds(p * hpp * nsrc, hpp * nsrc)], elsp)
                pltpu.sync_copy(
                    elerT_dst.at[pl.ds(8 * ndst + p * hpp * ndst,
                                       hpp * ndst)], ersp)
            pltpu.sync_copy(z2d.at[pl.ds(0, sl), pl.ds(0, kw)],
                            nsp.at[pl.ds(r0, sl)])
            pltpu.sync_copy(z1d.at[pl.ds(0, sl)], d0sp.at[pl.ds(r0, sl)])
            if hpp == 2:
                pltpu.sync_copy(z1d.at[pl.ds(0, sl)], d1sp.at[pl.ds(r0, sl)])
            plsc.subcore_barrier()

            def chunk(ch, _):
                b = wid * EPW + ch * CE
                pltpu.sync_copy(src_h.at[pl.ds(b, CE)], srcv)
                pltpu.sync_copy(dst_h.at[pl.ds(b, CE)], dstv)
                pltpu.sync_copy(kts[p].at[srcv], krows)
                for t in range(hpp):
                    h = p * hpp + t

                    def ib1(i, _):
                        sl16 = pl.ds(i * 16, 16)
                        idxb[sl16] = srcv[sl16] + t * nsrc
                        return 0

                    lax.fori_loop(0, CE // 16, ib1, 0)
                    pltpu.sync_copy(elsp.at[idxb], elv)

                    def ib2(i, _):
                        sl16 = pl.ds(i * 16, 16)
                        idxb[sl16] = dstv[sl16] + t * ndst
                        return 0

                    lax.fori_loop(0, CE // 16, ib2, 0)
                    pltpu.sync_copy(ersp.at[idxb], erv)
                    pltpu.sync_copy(ebt.at[h, pl.ds(b, CE)], ebv)
                    exv = exvs[t]

                    def sb(i, _):
                        sl16 = pl.ds(i * 16, 16)
                        v = elv[sl16] + erv[sl16] + ebv[sl16]
                        v = jnp.maximum(v, 0.2 * v)
                        exv[sl16] = jnp.exp(v)
                        return 0

                    lax.fori_loop(0, CE // 16, sb, 0)

                def mb(i, _):
                    a0 = exv0[pl.ds(i * 16, 16)]
                    if hpp == 2:
                        a1 = exv1[pl.ds(i * 16, 16)]
                    for j in range(16):
                        e = i * 16 + j
                        krows[e, pl.ds(0, 16)] = (
                            krows[e, pl.ds(0, 16)] * a0[j])
                        if hpp == 2:
                            krows[e, pl.ds(16, 16)] = (
                                krows[e, pl.ds(16, 16)] * a1[j])
                    return 0

                lax.fori_loop(0, CE // 16, mb, 0)
                pltpu.sync_copy(krows, nsp.at[dstv], add=True)
                pltpu.sync_copy(exv0, d0sp.at[dstv], add=True)
                if hpp == 2:
                    pltpu.sync_copy(exv1, d1sp.at[dstv], add=True)
                return 0

            lax.fori_loop(0, NCH, chunk, 0)
            plsc.subcore_barrier()
            pltpu.sync_copy(nsp.at[pl.ds(r0, sl)],
                            np_out.at[cid, pl.ds(r0, sl),
                                      pl.ds(p * kw, kw)])
            pltpu.sync_copy(d0sp.at[pl.ds(r0, sl)],
                            dp_out.at[cid, p * hpp, pl.ds(r0, sl)])
            if hpp == 2:
                pltpu.sync_copy(d1sp.at[pl.ds(r0, sl)],
                                dp_out.at[cid, p * hpp + 1, pl.ds(r0, sl)])
            plsc.subcore_barrier()

    return functools.partial(
        pl.kernel,
        out_type=(jax.ShapeDtypeStruct((2, npad, D), f32),
                  jax.ShapeDtypeStruct((2, 8, npad), f32)),
        mesh=plsc.VectorSubcoreMesh(**_MESH),
        compiler_params=pltpu.CompilerParams(use_tc_tiling_on_sc=False),
        scratch_types=scratch,
    )(body)


def _fold(W, a):
    return jnp.sum(W.reshape(D, H, DH) * a[None], axis=-1)


@jax.jit
def kernel(word_ids, edge_src, edge_dst, tffrac, sent_raw, embed, W_proj,
           TF_embed, W_edge, Wk1, Wq1, al1, ar1, w11, b11, w12, b12, g1, be1,
           Wk2, Wq2, al2, ar2, w21, b21, w22, b22, g2, be2, wh_w, wh_b):
    word_ids = word_ids.astype(i32)
    edge_src = edge_src.astype(i32)
    edge_dst = edge_dst.astype(i32)
    tffrac = tffrac.astype(i32)

    AL1, AR1 = _fold(Wk1, al1), _fold(Wq1, ar1)
    AL2, AR2 = _fold(Wk2, al2), _fold(Wq2, ar2)
    tfT = (TF_embed @ W_edge).T
    Wcat0 = jnp.concatenate([AL1, AR2], axis=1)
    Wpa = W_proj @ AR1
    P1 = jnp.concatenate([Wk2, AL2, AR1], axis=1)
    P2 = jnp.concatenate([Wk1, AL1, jnp.zeros((D, 8), f32)], axis=1)
    b11r, b12r = b11.reshape(1, FFN), b12.reshape(1, D)
    b21r, b22r = b21.reshape(1, FFN), b22.reshape(1, D)
    g1r, be1r = g1.reshape(1, D), be1.reshape(1, D)
    g2r, be2r = g2.reshape(1, D), be2.reshape(1, D)
    z2d = jnp.zeros((NW_SL, 32), f32)
    z1d = jnp.zeros((NW_SL,), f32)

    kt0, kt1, kt2, kt3, wt_voc = _embed_proj(embed, Wk1, Wcat0)
    wt_vocT = _transpose16(
        jnp.pad(wt_voc, ((0, VOC_PAD - VOC), (0, 0)))).reshape(-1)
    ser0 = _sent_proj(sent_raw, Wpa)
    selerT0 = _transpose16(
        jnp.pad(ser0, ((0, NS_PAD - NS), (0, 0)))).reshape(-1)
    ebt = _ebias(tffrac.reshape(1, E), tfT)
    k1t0, k1t1, k1t2, k1t3, welerT = _get_sc_word_gather()(
        word_ids, kt0, kt1, kt2, kt3, wt_vocT)

    pb_sent = _make_pass_b(NW_PAD, NS_PAD, NS_PAD, NS_SL, 2, 4)
    pb_word = _make_pass_b(NS_PAD, NW_PAD, NW_PAD, NW_SL, 1, 8)

    np1, dp1 = pb_sent(welerT, selerT0, ebt, k1t0, k1t1, k1t2, k1t3,
                       edge_src, edge_dst, z2d, z1d)
    c1 = _combine(np1, dp1, w11, b11r, w12, b12r, g1r, be1r, P1, nk=8)
    k2t = c1[:8]
    selerT = _transpose16(c1[8]).reshape(-1)

    np2, dp2 = pb_word(selerT, welerT, ebt, *k2t,
                       edge_dst, edge_src, z2d, z1d)
    c2 = _combine(np2, dp2, w21, b21r, w22, b22r, g2r, be2r, P2, nk=4)
    k3t = c2[:4]
    welerT2 = _transpose16(c2[4]).reshape(-1)

    np3, dp3 = pb_sent(welerT2, selerT, ebt, *k3t,
                       edge_src, edge_dst, z2d, z1d)
    result = _combine(np3, dp3, w11, b11r, w12, b12r, g1r, be1r,
                      wh_w, nk=0)
    return result[:NS] + wh_b
